# Initial kernel scaffold; baseline (speedup 1.0000x reference)
import jax
import jax.numpy as jnp
from jax.experimental import pallas as pl


def kernel(x, edge_index, W1, b1, W2, b2, perturb1, perturb2, perturb):
    raise NotImplementedError("write your pallas kernel here")


# speedup vs baseline: 9.1879x; 9.1879x over previous
import functools

import jax
import jax.numpy as jnp
from jax import lax
from jax.experimental import pallas as pl
from jax.experimental.pallas import tpu as pltpu
from jax.experimental.pallas import tpu_sc as plsc

NC = 2
NS = 16
NW = NC * NS
LANES = 16
EB = 128


def _mesh():
    return plsc.VectorSubcoreMesh(core_axis_name="c", subcore_axis_name="s")


def _make_deg_kernel(n_acc, n_out, d, k):
    rows_per_tile = n_acc // NS
    nzf = rows_per_tile // EB
    nzr = rows_per_tile - nzf * EB
    nch = k // CH

    @functools.partial(
        pl.kernel,
        out_type=jax.ShapeDtypeStruct((NC, n_out, d), jnp.float32),
        mesh=_mesh(),
        scratch_types=[
            pltpu.VMEM((CH, EB), jnp.int32),
            pltpu.VMEM((EB, d), jnp.float32),
            pltpu.VMEM_SHARED((n_acc, d), jnp.float32),
        ],
    )
    def deg_kernel(dst_hbm, out_hbm, dst_v, ones_v, acc):
        c = lax.axis_index("c")
        s = lax.axis_index("s")
        wid = c * NS + s
        base = s * rows_per_tile

        def fillz(r, _):
            for t in range(d // LANES):
                ones_v[r, pl.ds(t * LANES, LANES)] = jnp.zeros((LANES,), jnp.float32)
            return 0
        lax.fori_loop(0, EB, fillz, 0)

        def zero(i, _):
            pltpu.sync_copy(ones_v, acc.at[pl.ds(base + i * EB, EB)])
            return 0
        lax.fori_loop(0, nzf, zero, 0)
        if nzr:
            pltpu.sync_copy(ones_v.at[pl.ds(0, nzr)],
                            acc.at[pl.ds(base + nzf * EB, nzr)])

        def fill1(r, _):
            for t in range(d // LANES):
                ones_v[r, pl.ds(t * LANES, LANES)] = jnp.full((LANES,), 1.0, jnp.float32)
            return 0
        lax.fori_loop(0, EB, fill1, 0)
        plsc.subcore_barrier()

        def chunk(ci, _):
            pltpu.sync_copy(dst_hbm.at[wid, pl.ds(ci * CH, CH)], dst_v)

            def step(j, _):
                pltpu.sync_copy(ones_v, acc.at[dst_v.at[j]], add=True)
                return 0
            lax.fori_loop(0, CH, step, 0)
            return 0
        lax.fori_loop(0, nch, chunk, 0)
        plsc.subcore_barrier()

        pltpu.sync_copy(acc.at[pl.ds(base, rows_per_tile)],
                        out_hbm.at[c, pl.ds(base, rows_per_tile)])

    return deg_kernel


CH = 16


def _make_agg_kernel(n_acc, n_out, d, k):
    rows_per_tile = n_acc // NS
    nzf = rows_per_tile // EB
    nzr = rows_per_tile - nzf * EB
    nch = k // CH

    @functools.partial(
        pl.kernel,
        out_type=jax.ShapeDtypeStruct((NC, n_out, d), jnp.float32),
        mesh=_mesh(),
        scratch_types=[
            pltpu.VMEM((CH, EB), jnp.int32),
            pltpu.VMEM((CH, EB), jnp.int32),
            pltpu.VMEM((2, EB, d), jnp.float32),
            pltpu.VMEM_SHARED((n_acc, d), jnp.float32),
            pltpu.SemaphoreType.DMA,
            pltpu.SemaphoreType.DMA,
        ],
    )
    def agg_kernel(y_hbm, src_hbm, dst_hbm, out_hbm,
                   src_v, dst_v, rows_v, acc, sem0, sem1):
        c = lax.axis_index("c")
        s = lax.axis_index("s")
        wid = c * NS + s
        base = s * rows_per_tile

        def fill(r, _):
            for t in range(d // LANES):
                rows_v[0, r, pl.ds(t * LANES, LANES)] = jnp.zeros((LANES,), jnp.float32)
            return 0
        lax.fori_loop(0, EB, fill, 0)

        def zero(i, _):
            pltpu.sync_copy(rows_v.at[0], acc.at[pl.ds(base + i * EB, EB)])
            return 0
        lax.fori_loop(0, nzf, zero, 0)
        if nzr:
            pltpu.sync_copy(rows_v.at[0, pl.ds(0, nzr)],
                            acc.at[pl.ds(base + nzf * EB, nzr)])
        plsc.subcore_barrier()

        def chunk(ci, _):
            pltpu.sync_copy(src_hbm.at[wid, pl.ds(ci * CH, CH)], src_v)
            pltpu.sync_copy(dst_hbm.at[wid, pl.ds(ci * CH, CH)], dst_v)
            pltpu.async_copy(y_hbm.at[src_v.at[0]], rows_v.at[0], sem0)

            def step(jj, _):
                j0 = 2 * jj
                j1 = j0 + 1
                pltpu.async_copy(y_hbm.at[src_v.at[j1]], rows_v.at[1], sem1)
                pltpu.make_async_copy(y_hbm.at[src_v.at[j0]], rows_v.at[0], sem0).wait()
                pltpu.sync_copy(rows_v.at[0], acc.at[dst_v.at[j0]], add=True)

                @pl.when(jj + 1 < CH // 2)
                def _():
                    pltpu.async_copy(y_hbm.at[src_v.at[j0 + 2]], rows_v.at[0], sem0)

                pltpu.make_async_copy(y_hbm.at[src_v.at[j1]], rows_v.at[1], sem1).wait()
                pltpu.sync_copy(rows_v.at[1], acc.at[dst_v.at[j1]], add=True)
                return 0
            lax.fori_loop(0, CH // 2, step, 0)
            return 0
        lax.fori_loop(0, nch, chunk, 0)
        plsc.subcore_barrier()

        pltpu.sync_copy(acc.at[pl.ds(base, rows_per_tile)],
                        out_hbm.at[c, pl.ds(base, rows_per_tile)])

    return agg_kernel


def _dinv_block(degp):
    deg = degp[0] + degp[1] + 1.0
    return lax.rsqrt(jnp.maximum(deg, 1.0))[:, 0:1]


def _tc_pre(x_ref, w_ref, degp_ref, y_ref):
    dinv = _dinv_block(degp_ref[...])
    h = jnp.dot(x_ref[...], w_ref[...], preferred_element_type=jnp.float32)
    y_ref[...] = h * dinv


def _tc_mid(g_ref, aggp_ref, y_ref, degp_ref, b_ref, p_ref, w_ref, y2_ref):
    dinv = _dinv_block(degp_ref[...])
    t = (aggp_ref[0] + aggp_ref[1] + y_ref[...]) * dinv
    t = t + b_ref[...] + g_ref[0] * p_ref[...]
    z = jnp.maximum(t, 0.0)
    y2_ref[...] = jnp.dot(z, w_ref[...], preferred_element_type=jnp.float32) * dinv


def _tc_post(g_ref, aggp_ref, y_ref, degp_ref, b_ref, p_ref, out_ref):
    dinv = _dinv_block(degp_ref[...])
    t = (aggp_ref[0] + aggp_ref[1] + y_ref[...]) * dinv
    out_ref[...] = t + b_ref[...] + g_ref[0] * p_ref[...]


def kernel(x, edge_index, W1, b1, W2, b2, perturb1, perturb2, perturb):
    n, d = x.shape
    e = edge_index.shape[1]

    bn = 1024
    n_pad = ((n + bn - 1) // bn) * bn
    n_acc = ((n + 1 + NS * 8 - 1) // (NS * 8)) * (NS * 8)
    k = -(-e // (NW * EB))
    k = ((k + CH - 1) // CH) * CH
    e_pad = NW * k * EB

    src = jnp.concatenate([edge_index[0], jnp.zeros((e_pad - e,), jnp.int32)])
    dst = jnp.concatenate([edge_index[1],
                           jnp.full((e_pad - e,), n, jnp.int32)])
    src_t = src.reshape(NW, k, EB)
    dst_t = dst.reshape(NW, k, EB)
    x_p = jnp.pad(x, ((0, n_pad - n), (0, 0)))
    p1_p = jnp.pad(perturb1, ((0, n_pad - n), (0, 0)))
    p2_p = jnp.pad(perturb2, ((0, n_pad - n), (0, 0)))
    b1_r = b1.reshape(1, d)
    b2_r = b2.reshape(1, d)
    g = jnp.asarray(perturb, jnp.float32).reshape(1)

    grid = (n_pad // bn,)
    row_spec = pl.BlockSpec((bn, d), lambda i: (i, 0))
    w_spec = pl.BlockSpec((d, d), lambda i: (0, 0))
    degp_spec = pl.BlockSpec((NC, bn, d), lambda i: (0, i, 0))
    aggp_spec = pl.BlockSpec((NC, bn, d), lambda i: (0, i, 0))
    b_spec = pl.BlockSpec((1, d), lambda i: (0, 0))
    g_spec = pl.BlockSpec(memory_space=pltpu.MemorySpace.SMEM)
    row_out = jax.ShapeDtypeStruct((n_pad, d), jnp.float32)

    deg_kernel = _make_deg_kernel(n_acc, n_pad, d, k)
    agg_kernel = _make_agg_kernel(n_acc, n_pad, d, k)

    degp = deg_kernel(dst_t)

    y1 = pl.pallas_call(
        _tc_pre, grid=grid,
        in_specs=[row_spec, w_spec, degp_spec],
        out_specs=row_spec, out_shape=row_out,
    )(x_p, W1, degp)

    agg1 = agg_kernel(y1, src_t, dst_t)

    y2 = pl.pallas_call(
        _tc_mid, grid=grid,
        in_specs=[g_spec, aggp_spec, row_spec, degp_spec, b_spec, row_spec, w_spec],
        out_specs=row_spec, out_shape=row_out,
    )(g, agg1, y1, degp, b1_r, p1_p, W2)

    agg2 = agg_kernel(y2, src_t, dst_t)

    out = pl.pallas_call(
        _tc_post, grid=grid,
        in_specs=[g_spec, aggp_spec, row_spec, degp_spec, b_spec, row_spec],
        out_specs=row_spec, out_shape=row_out,
    )(g, agg2, y2, degp, b2_r, p2_p)

    return out[:n]
